# baseline (speedup 1.0000x reference)
import functools

import jax
import jax.numpy as jnp
from jax.experimental import pallas as pl
from jax.experimental.pallas import tpu as pltpu


def _div_tile(n, cap):
    t = 8
    for c in range(8, min(n, cap) + 1, 8):
        if n % c == 0:
            t = c
    return t


def _agg_body(s1_ref, s2_ref, s3_ref, w_ref, adj_ref, adj3_ref, b12_ref,
              b3_ref, a_ref, mskc_ref, h12_ref, h3_ref, csum_ref,
              f12_scr, f3_scr, *, h):
    i = pl.program_id(0)

    @pl.when(i == 0)
    def _():
        w = w_ref[...]
        f12_scr[:, :h] = jnp.dot(s1_ref[...], w,
                                 preferred_element_type=jnp.float32)
        f12_scr[:, h:] = jnp.dot(s2_ref[...], w,
                                 preferred_element_type=jnp.float32)
        f3_scr[...] = jnp.dot(s3_ref[...], w,
                              preferred_element_type=jnp.float32)

    a = a_ref[0, 0]
    h12 = jnp.dot(adj_ref[...], f12_scr[...],
                  preferred_element_type=jnp.float32) + b12_ref[...]
    h12 = jnp.where(h12 >= 0, h12, a * h12)
    h12_ref[...] = h12
    h3v = jnp.dot(adj3_ref[...], f3_scr[...],
                  preferred_element_type=jnp.float32) + b3_ref[...]
    h3_ref[...] = jnp.where(h3v >= 0, h3v, a * h3v)
    csum_ref[0, 0, :] = jnp.sum(h12[:, :h] * mskc_ref[...], axis=0)


def _disc_body(h12_ref, h3_ref, csum_ref, msk_ref, wt_ref, bd_ref,
               sb1_ref, sb2a_ref, sb2b_ref, s1_ref, s2a_ref, s2b_ref, *, h):
    csum = jnp.sum(csum_ref[...], axis=0)
    msum = jnp.sum(msk_ref[...])
    c = jax.nn.sigmoid(csum / msum)
    v = jnp.dot(c, wt_ref[...], preferred_element_type=jnp.float32)
    bd = bd_ref[0, 0]
    h12 = h12_ref[...]
    s1 = jnp.sum(h12[:, :h] * v, axis=1, keepdims=True)
    s2a = jnp.sum(h12[:, h:] * v, axis=1, keepdims=True)
    s2b = jnp.sum(h3_ref[...] * v, axis=1, keepdims=True)
    s1_ref[...] = s1 + bd + sb1_ref[...]
    s2a_ref[...] = s2a + bd + sb2a_ref[...]
    s2b_ref[...] = s2b + bd + sb2b_ref[...]


def kernel(seq1, seq2, seq3, adj, adj3, sparse, msk, samp_bias1, samp_bias2,
           W_gcn, b_gcn, a_prelu, W_disc, b_disc):
    n, d = seq1.shape[1], seq1.shape[2]
    h = W_gcn.shape[1]
    f32 = jnp.float32

    s1 = seq1[0]
    s2 = seq2[0]
    s3 = seq3[0]
    a2 = adj[0]
    a3 = adj3[0]

    t_i = _div_tile(n, 80)
    n_i = n // t_i
    b12 = jnp.concatenate([b_gcn, b_gcn]).reshape(1, 2 * h)
    b3 = b_gcn.reshape(1, h)
    a_s = a_prelu.reshape(1, 1)
    mskc = msk.reshape(n, 1)

    h12, h3, csum_p = pl.pallas_call(
        functools.partial(_agg_body, h=h),
        grid=(n_i,),
        in_specs=[
            pl.BlockSpec((n, d), lambda i: (0, 0)),
            pl.BlockSpec((n, d), lambda i: (0, 0)),
            pl.BlockSpec((n, d), lambda i: (0, 0)),
            pl.BlockSpec((d, h), lambda i: (0, 0)),
            pl.BlockSpec((t_i, n), lambda i: (i, 0)),
            pl.BlockSpec((t_i, n), lambda i: (i, 0)),
            pl.BlockSpec((1, 2 * h), lambda i: (0, 0)),
            pl.BlockSpec((1, h), lambda i: (0, 0)),
            pl.BlockSpec((1, 1), lambda i: (0, 0)),
            pl.BlockSpec((t_i, 1), lambda i: (i, 0)),
        ],
        out_specs=[
            pl.BlockSpec((t_i, 2 * h), lambda i: (i, 0)),
            pl.BlockSpec((t_i, h), lambda i: (i, 0)),
            pl.BlockSpec((1, 1, h), lambda i: (i, 0, 0)),
        ],
        out_shape=[
            jax.ShapeDtypeStruct((n, 2 * h), f32),
            jax.ShapeDtypeStruct((n, h), f32),
            jax.ShapeDtypeStruct((n_i, 1, h), f32),
        ],
        scratch_shapes=[
            pltpu.VMEM((n, 2 * h), f32),
            pltpu.VMEM((n, h), f32),
        ],
    )(s1, s2, s3, W_gcn, a2, a3, b12, b3, a_s, mskc)

    t_n = _div_tile(n, 2000)
    wt = W_disc.T
    bd = b_disc.reshape(1, 1)
    sb1 = samp_bias1.reshape(n, 1)
    sb2a = samp_bias2[:, :n].reshape(n, 1)
    sb2b = samp_bias2[:, n:].reshape(n, 1)

    s_1, s_2a, s_2b = pl.pallas_call(
        functools.partial(_disc_body, h=h),
        grid=(n // t_n,),
        in_specs=[
            pl.BlockSpec((t_n, 2 * h), lambda i: (i, 0)),
            pl.BlockSpec((t_n, h), lambda i: (i, 0)),
            pl.BlockSpec((n_i, 1, h), lambda i: (0, 0, 0)),
            pl.BlockSpec((1, n), lambda i: (0, 0)),
            pl.BlockSpec((h, h), lambda i: (0, 0)),
            pl.BlockSpec((1, 1), lambda i: (0, 0)),
            pl.BlockSpec((t_n, 1), lambda i: (i, 0)),
            pl.BlockSpec((t_n, 1), lambda i: (i, 0)),
            pl.BlockSpec((t_n, 1), lambda i: (i, 0)),
        ],
        out_specs=[
            pl.BlockSpec((t_n, 1), lambda i: (i, 0)),
            pl.BlockSpec((t_n, 1), lambda i: (i, 0)),
            pl.BlockSpec((t_n, 1), lambda i: (i, 0)),
        ],
        out_shape=[
            jax.ShapeDtypeStruct((n, 1), f32),
            jax.ShapeDtypeStruct((n, 1), f32),
            jax.ShapeDtypeStruct((n, 1), f32),
        ],
    )(h12, h3, csum_p, msk, wt, bd, sb1, sb2a, sb2b)

    return jnp.concatenate([s_1, s_2a, s_2b], axis=0).reshape(1, 3 * n)


# speedup vs baseline: 1.1008x; 1.1008x over previous
import functools

import jax
import jax.numpy as jnp
from jax.experimental import pallas as pl


def _div_tile(n, cap):
    t = 8
    for c in range(8, min(n, cap) + 1, 8):
        if n % c == 0:
            t = c
    return t


def _fts_body(s1_ref, s2_ref, s3_ref, w_ref, f12_ref, f3_ref):
    w = w_ref[...]
    h = w.shape[1]
    f12_ref[:, :h] = jnp.dot(s1_ref[...], w, preferred_element_type=jnp.float32)
    f12_ref[:, h:] = jnp.dot(s2_ref[...], w, preferred_element_type=jnp.float32)
    f3_ref[...] = jnp.dot(s3_ref[...], w, preferred_element_type=jnp.float32)


def _agg_body(adj_ref, adj3_ref, f12_ref, f3_ref, b12_ref, b3_ref, a_ref,
              mskc_ref, h12_ref, h3_ref, csum_ref, *, h):
    a = a_ref[0, 0]
    h12 = jnp.dot(adj_ref[...], f12_ref[...],
                  preferred_element_type=jnp.float32) + b12_ref[...]
    h12 = jnp.where(h12 >= 0, h12, a * h12)
    h12_ref[...] = h12.astype(h12_ref.dtype)
    h3v = jnp.dot(adj3_ref[...], f3_ref[...],
                  preferred_element_type=jnp.float32) + b3_ref[...]
    h3v = jnp.where(h3v >= 0, h3v, a * h3v)
    h3_ref[...] = h3v.astype(h3_ref.dtype)
    csum_ref[0, 0, :] = jnp.sum(h12[:, :h] * mskc_ref[...], axis=0)


def _disc_body(h12_ref, h3_ref, csum_ref, msk_ref, wt_ref, bd_ref,
               sb1_ref, sb2a_ref, sb2b_ref, s1_ref, s2a_ref, s2b_ref, *, h):
    csum = jnp.sum(csum_ref[...], axis=0)
    msum = jnp.sum(msk_ref[...])
    c = jax.nn.sigmoid(csum / msum)
    v = jnp.dot(c, wt_ref[...], preferred_element_type=jnp.float32)
    bd = bd_ref[0, 0]
    h12 = h12_ref[...].astype(jnp.float32)
    s1 = jnp.sum(h12[:, :h] * v, axis=1, keepdims=True)
    s2a = jnp.sum(h12[:, h:] * v, axis=1, keepdims=True)
    s2b = jnp.sum(h3_ref[...].astype(jnp.float32) * v, axis=1, keepdims=True)
    s1_ref[...] = s1 + bd + sb1_ref[...]
    s2a_ref[...] = s2a + bd + sb2a_ref[...]
    s2b_ref[...] = s2b + bd + sb2b_ref[...]


def kernel(seq1, seq2, seq3, adj, adj3, sparse, msk, samp_bias1, samp_bias2,
           W_gcn, b_gcn, a_prelu, W_disc, b_disc):
    n, d = seq1.shape[1], seq1.shape[2]
    h = W_gcn.shape[1]
    f32 = jnp.float32
    bf16 = jnp.bfloat16

    s1 = seq1[0]
    s2 = seq2[0]
    s3 = seq3[0]
    a2 = adj[0]
    a3 = adj3[0]

    t1 = _div_tile(n, 2000)
    f12, f3 = pl.pallas_call(
        _fts_body,
        grid=(n // t1,),
        in_specs=[
            pl.BlockSpec((t1, d), lambda i: (i, 0)),
            pl.BlockSpec((t1, d), lambda i: (i, 0)),
            pl.BlockSpec((t1, d), lambda i: (i, 0)),
            pl.BlockSpec((d, h), lambda i: (0, 0)),
        ],
        out_specs=[
            pl.BlockSpec((t1, 2 * h), lambda i: (i, 0)),
            pl.BlockSpec((t1, h), lambda i: (i, 0)),
        ],
        out_shape=[
            jax.ShapeDtypeStruct((n, 2 * h), f32),
            jax.ShapeDtypeStruct((n, h), f32),
        ],
    )(s1, s2, s3, W_gcn)

    t_i = _div_tile(n, 200)
    n_i = n // t_i
    b12 = jnp.concatenate([b_gcn, b_gcn]).reshape(1, 2 * h)
    b3 = b_gcn.reshape(1, h)
    a_s = a_prelu.reshape(1, 1)
    mskc = msk.reshape(n, 1)

    h12, h3, csum_p = pl.pallas_call(
        functools.partial(_agg_body, h=h),
        grid=(n_i,),
        in_specs=[
            pl.BlockSpec((t_i, n), lambda i: (i, 0)),
            pl.BlockSpec((t_i, n), lambda i: (i, 0)),
            pl.BlockSpec((n, 2 * h), lambda i: (0, 0)),
            pl.BlockSpec((n, h), lambda i: (0, 0)),
            pl.BlockSpec((1, 2 * h), lambda i: (0, 0)),
            pl.BlockSpec((1, h), lambda i: (0, 0)),
            pl.BlockSpec((1, 1), lambda i: (0, 0)),
            pl.BlockSpec((t_i, 1), lambda i: (i, 0)),
        ],
        out_specs=[
            pl.BlockSpec((t_i, 2 * h), lambda i: (i, 0)),
            pl.BlockSpec((t_i, h), lambda i: (i, 0)),
            pl.BlockSpec((1, 1, h), lambda i: (i, 0, 0)),
        ],
        out_shape=[
            jax.ShapeDtypeStruct((n, 2 * h), bf16),
            jax.ShapeDtypeStruct((n, h), bf16),
            jax.ShapeDtypeStruct((n_i, 1, h), f32),
        ],
    )(a2, a3, f12, f3, b12, b3, a_s, mskc)

    t_n = _div_tile(n, 2000)
    wt = W_disc.T
    bd = b_disc.reshape(1, 1)
    sb1 = samp_bias1.reshape(n, 1)
    sb2a = samp_bias2[:, :n].reshape(n, 1)
    sb2b = samp_bias2[:, n:].reshape(n, 1)

    s_1, s_2a, s_2b = pl.pallas_call(
        functools.partial(_disc_body, h=h),
        grid=(n // t_n,),
        in_specs=[
            pl.BlockSpec((t_n, 2 * h), lambda i: (i, 0)),
            pl.BlockSpec((t_n, h), lambda i: (i, 0)),
            pl.BlockSpec((n_i, 1, h), lambda i: (0, 0, 0)),
            pl.BlockSpec((1, n), lambda i: (0, 0)),
            pl.BlockSpec((h, h), lambda i: (0, 0)),
            pl.BlockSpec((1, 1), lambda i: (0, 0)),
            pl.BlockSpec((t_n, 1), lambda i: (i, 0)),
            pl.BlockSpec((t_n, 1), lambda i: (i, 0)),
            pl.BlockSpec((t_n, 1), lambda i: (i, 0)),
        ],
        out_specs=[
            pl.BlockSpec((t_n, 1), lambda i: (i, 0)),
            pl.BlockSpec((t_n, 1), lambda i: (i, 0)),
            pl.BlockSpec((t_n, 1), lambda i: (i, 0)),
        ],
        out_shape=[
            jax.ShapeDtypeStruct((n, 1), f32),
            jax.ShapeDtypeStruct((n, 1), f32),
            jax.ShapeDtypeStruct((n, 1), f32),
        ],
    )(h12, h3, csum_p, msk, wt, bd, sb1, sb2a, sb2b)

    return jnp.concatenate([s_1, s_2a, s_2b], axis=0).reshape(1, 3 * n)


# speedup vs baseline: 1.1210x; 1.0184x over previous
import functools

import jax
import jax.numpy as jnp
from jax.experimental import pallas as pl
from jax.experimental.pallas import tpu as pltpu


def _div_tile(n, cap):
    t = 8
    for c in range(8, min(n, cap) + 1, 8):
        if n % c == 0:
            t = c
    return t


def _agg12_body(s1_ref, s2_ref, w_ref, adj_ref, b12_ref, a_ref, mskc_ref,
                h12_ref, csum_ref, f12_ref, *, h):
    @pl.when(pl.program_id(0) == 0)
    def _():
        w = w_ref[...]
        f12_ref[:, :h] = jnp.dot(s1_ref[...], w,
                                 preferred_element_type=jnp.float32)
        f12_ref[:, h:] = jnp.dot(s2_ref[...], w,
                                 preferred_element_type=jnp.float32)

    a = a_ref[0, 0]
    h12 = jnp.dot(adj_ref[...], f12_ref[...],
                  preferred_element_type=jnp.float32) + b12_ref[...]
    h12 = jnp.where(h12 >= 0, h12, a * h12)
    h12_ref[...] = h12.astype(h12_ref.dtype)
    csum_ref[0, 0, :] = jnp.sum(h12[:, :h] * mskc_ref[...], axis=0)


def _agg3_body(s3_ref, w_ref, adj3_ref, b3_ref, a_ref, h3_ref, f3_ref):
    @pl.when(pl.program_id(0) == 0)
    def _():
        f3_ref[...] = jnp.dot(s3_ref[...], w_ref[...],
                              preferred_element_type=jnp.float32)

    a = a_ref[0, 0]
    h3v = jnp.dot(adj3_ref[...], f3_ref[...],
                  preferred_element_type=jnp.float32) + b3_ref[...]
    h3_ref[...] = jnp.where(h3v >= 0, h3v, a * h3v).astype(h3_ref.dtype)


def _disc_body(h12_ref, h3_ref, csum_ref, msk_ref, wt_ref, bd_ref,
               sb1_ref, sb2a_ref, sb2b_ref, s1_ref, s2a_ref, s2b_ref, *, h):
    csum = jnp.sum(csum_ref[...], axis=0)
    msum = jnp.sum(msk_ref[...])
    c = jax.nn.sigmoid(csum / msum)
    v = jnp.dot(c, wt_ref[...], preferred_element_type=jnp.float32)
    bd = bd_ref[0, 0]
    h12 = h12_ref[...].astype(jnp.float32)
    s1 = jnp.sum(h12[:, :h] * v, axis=1, keepdims=True)
    s2a = jnp.sum(h12[:, h:] * v, axis=1, keepdims=True)
    s2b = jnp.sum(h3_ref[...].astype(jnp.float32) * v, axis=1, keepdims=True)
    s1_ref[...] = s1 + bd + sb1_ref[...]
    s2a_ref[...] = s2a + bd + sb2a_ref[...]
    s2b_ref[...] = s2b + bd + sb2b_ref[...]


def kernel(seq1, seq2, seq3, adj, adj3, sparse, msk, samp_bias1, samp_bias2,
           W_gcn, b_gcn, a_prelu, W_disc, b_disc):
    n, d = seq1.shape[1], seq1.shape[2]
    h = W_gcn.shape[1]
    f32 = jnp.float32
    bf16 = jnp.bfloat16

    s1 = seq1[0]
    s2 = seq2[0]
    s3 = seq3[0]
    a2 = adj[0]
    a3 = adj3[0]

    t_i = _div_tile(n, 400)
    n_i = n // t_i
    b12 = jnp.concatenate([b_gcn, b_gcn]).reshape(1, 2 * h)
    b3 = b_gcn.reshape(1, h)
    a_s = a_prelu.reshape(1, 1)
    mskc = msk.reshape(n, 1)

    h12, csum_p = pl.pallas_call(
        functools.partial(_agg12_body, h=h),
        grid=(n_i,),
        in_specs=[
            pl.BlockSpec((n, d), lambda i: (0, 0)),
            pl.BlockSpec((n, d), lambda i: (0, 0)),
            pl.BlockSpec((d, h), lambda i: (0, 0)),
            pl.BlockSpec((t_i, n), lambda i: (i, 0)),
            pl.BlockSpec((1, 2 * h), lambda i: (0, 0)),
            pl.BlockSpec((1, 1), lambda i: (0, 0)),
            pl.BlockSpec((t_i, 1), lambda i: (i, 0)),
        ],
        out_specs=[
            pl.BlockSpec((t_i, 2 * h), lambda i: (i, 0)),
            pl.BlockSpec((1, 1, h), lambda i: (i, 0, 0)),
        ],
        out_shape=[
            jax.ShapeDtypeStruct((n, 2 * h), bf16),
            jax.ShapeDtypeStruct((n_i, 1, h), f32),
        ],
        scratch_shapes=[
            pltpu.VMEM((n, 2 * h), f32),
        ],
    )(s1, s2, W_gcn, a2, b12, a_s, mskc)

    h3 = pl.pallas_call(
        _agg3_body,
        grid=(n_i,),
        in_specs=[
            pl.BlockSpec((n, d), lambda i: (0, 0)),
            pl.BlockSpec((d, h), lambda i: (0, 0)),
            pl.BlockSpec((t_i, n), lambda i: (i, 0)),
            pl.BlockSpec((1, h), lambda i: (0, 0)),
            pl.BlockSpec((1, 1), lambda i: (0, 0)),
        ],
        out_specs=pl.BlockSpec((t_i, h), lambda i: (i, 0)),
        out_shape=jax.ShapeDtypeStruct((n, h), bf16),
        scratch_shapes=[
            pltpu.VMEM((n, h), f32),
        ],
    )(s3, W_gcn, a3, b3, a_s)

    t_n = _div_tile(n, 2000)
    wt = W_disc.T
    bd = b_disc.reshape(1, 1)
    sb1 = samp_bias1.reshape(n, 1)
    sb2a = samp_bias2[:, :n].reshape(n, 1)
    sb2b = samp_bias2[:, n:].reshape(n, 1)

    s_1, s_2a, s_2b = pl.pallas_call(
        functools.partial(_disc_body, h=h),
        grid=(n // t_n,),
        in_specs=[
            pl.BlockSpec((t_n, 2 * h), lambda i: (i, 0)),
            pl.BlockSpec((t_n, h), lambda i: (i, 0)),
            pl.BlockSpec((n_i, 1, h), lambda i: (0, 0, 0)),
            pl.BlockSpec((1, n), lambda i: (0, 0)),
            pl.BlockSpec((h, h), lambda i: (0, 0)),
            pl.BlockSpec((1, 1), lambda i: (0, 0)),
            pl.BlockSpec((t_n, 1), lambda i: (i, 0)),
            pl.BlockSpec((t_n, 1), lambda i: (i, 0)),
            pl.BlockSpec((t_n, 1), lambda i: (i, 0)),
        ],
        out_specs=[
            pl.BlockSpec((t_n, 1), lambda i: (i, 0)),
            pl.BlockSpec((t_n, 1), lambda i: (i, 0)),
            pl.BlockSpec((t_n, 1), lambda i: (i, 0)),
        ],
        out_shape=[
            jax.ShapeDtypeStruct((n, 1), f32),
            jax.ShapeDtypeStruct((n, 1), f32),
            jax.ShapeDtypeStruct((n, 1), f32),
        ],
    )(h12, h3, csum_p, msk, wt, bd, sb1, sb2a, sb2b)

    return jnp.concatenate([s_1, s_2a, s_2b], axis=0).reshape(1, 3 * n)


# speedup vs baseline: 1.1313x; 1.0091x over previous
import functools

import jax
import jax.numpy as jnp
from jax.experimental import pallas as pl
from jax.experimental.pallas import tpu as pltpu


def _div_tile(n, cap):
    t = 8
    for c in range(8, min(n, cap) + 1, 8):
        if n % c == 0:
            t = c
    return t


def _agg12_body(s1_ref, s2_ref, w_ref, adj_ref, b12_ref, a_ref, mskc_ref,
                h12_ref, csum_ref, f12_ref, *, h):
    @pl.when(pl.program_id(0) == 0)
    def _():
        w = w_ref[...]
        f12_ref[:, :h] = jnp.dot(s1_ref[...], w,
                                 preferred_element_type=jnp.float32)
        f12_ref[:, h:] = jnp.dot(s2_ref[...], w,
                                 preferred_element_type=jnp.float32)

    a = a_ref[0, 0]
    h12 = jnp.dot(adj_ref[...], f12_ref[...],
                  preferred_element_type=jnp.float32) + b12_ref[...]
    h12 = jnp.where(h12 >= 0, h12, a * h12)
    h12_ref[...] = h12.astype(h12_ref.dtype)
    csum_ref[0, 0, :] = jnp.sum(h12[:, :h] * mskc_ref[...], axis=0)


def _agg3d_body(s3_ref, w_ref, adj3_ref, h12_ref, csum_ref, msk_ref, wt_ref,
                b3_ref, a_ref, bd_ref, sb1_ref, sb2a_ref, sb2b_ref,
                s1_ref, s2a_ref, s2b_ref, f3_scr, v_scr, *, h):
    @pl.when(pl.program_id(0) == 0)
    def _():
        f3_scr[...] = jnp.dot(s3_ref[...], w_ref[...],
                              preferred_element_type=jnp.float32)
        csum = jnp.sum(csum_ref[...], axis=0)
        msum = jnp.sum(msk_ref[...])
        c = jax.nn.sigmoid(csum / msum)
        v_scr[...] = jnp.dot(c, wt_ref[...],
                             preferred_element_type=jnp.float32)

    a = a_ref[0, 0]
    bd = bd_ref[0, 0]
    v = v_scr[...]
    h3 = jnp.dot(adj3_ref[...], f3_scr[...],
                 preferred_element_type=jnp.float32) + b3_ref[...]
    h3 = jnp.where(h3 >= 0, h3, a * h3)
    s2b_ref[...] = jnp.sum(h3 * v, axis=1, keepdims=True) + bd + sb2b_ref[...]
    h12 = h12_ref[...].astype(jnp.float32)
    s1_ref[...] = (jnp.sum(h12[:, :h] * v, axis=1, keepdims=True)
                   + bd + sb1_ref[...])
    s2a_ref[...] = (jnp.sum(h12[:, h:] * v, axis=1, keepdims=True)
                    + bd + sb2a_ref[...])


def kernel(seq1, seq2, seq3, adj, adj3, sparse, msk, samp_bias1, samp_bias2,
           W_gcn, b_gcn, a_prelu, W_disc, b_disc):
    n, d = seq1.shape[1], seq1.shape[2]
    h = W_gcn.shape[1]
    f32 = jnp.float32
    bf16 = jnp.bfloat16

    s1 = seq1[0]
    s2 = seq2[0]
    s3 = seq3[0]
    a2 = adj[0]
    a3 = adj3[0]

    t_i = _div_tile(n, 400)
    n_i = n // t_i
    b12 = jnp.concatenate([b_gcn, b_gcn]).reshape(1, 2 * h)
    b3 = b_gcn.reshape(1, h)
    a_s = a_prelu.reshape(1, 1)
    mskc = msk.reshape(n, 1)

    h12, csum_p = pl.pallas_call(
        functools.partial(_agg12_body, h=h),
        grid=(n_i,),
        in_specs=[
            pl.BlockSpec((n, d), lambda i: (0, 0)),
            pl.BlockSpec((n, d), lambda i: (0, 0)),
            pl.BlockSpec((d, h), lambda i: (0, 0)),
            pl.BlockSpec((t_i, n), lambda i: (i, 0)),
            pl.BlockSpec((1, 2 * h), lambda i: (0, 0)),
            pl.BlockSpec((1, 1), lambda i: (0, 0)),
            pl.BlockSpec((t_i, 1), lambda i: (i, 0)),
        ],
        out_specs=[
            pl.BlockSpec((t_i, 2 * h), lambda i: (i, 0)),
            pl.BlockSpec((1, 1, h), lambda i: (i, 0, 0)),
        ],
        out_shape=[
            jax.ShapeDtypeStruct((n, 2 * h), bf16),
            jax.ShapeDtypeStruct((n_i, 1, h), f32),
        ],
        scratch_shapes=[
            pltpu.VMEM((n, 2 * h), f32),
        ],
    )(s1, s2, W_gcn, a2, b12, a_s, mskc)

    wt = W_disc.T
    bd = b_disc.reshape(1, 1)
    sb1 = samp_bias1.reshape(n, 1)
    sb2a = samp_bias2[:, :n].reshape(n, 1)
    sb2b = samp_bias2[:, n:].reshape(n, 1)

    s_1, s_2a, s_2b = pl.pallas_call(
        functools.partial(_agg3d_body, h=h),
        grid=(n_i,),
        in_specs=[
            pl.BlockSpec((n, d), lambda i: (0, 0)),
            pl.BlockSpec((d, h), lambda i: (0, 0)),
            pl.BlockSpec((t_i, n), lambda i: (i, 0)),
            pl.BlockSpec((t_i, 2 * h), lambda i: (i, 0)),
            pl.BlockSpec((n_i, 1, h), lambda i: (0, 0, 0)),
            pl.BlockSpec((1, n), lambda i: (0, 0)),
            pl.BlockSpec((h, h), lambda i: (0, 0)),
            pl.BlockSpec((1, h), lambda i: (0, 0)),
            pl.BlockSpec((1, 1), lambda i: (0, 0)),
            pl.BlockSpec((1, 1), lambda i: (0, 0)),
            pl.BlockSpec((t_i, 1), lambda i: (i, 0)),
            pl.BlockSpec((t_i, 1), lambda i: (i, 0)),
            pl.BlockSpec((t_i, 1), lambda i: (i, 0)),
        ],
        out_specs=[
            pl.BlockSpec((t_i, 1), lambda i: (i, 0)),
            pl.BlockSpec((t_i, 1), lambda i: (i, 0)),
            pl.BlockSpec((t_i, 1), lambda i: (i, 0)),
        ],
        out_shape=[
            jax.ShapeDtypeStruct((n, 1), f32),
            jax.ShapeDtypeStruct((n, 1), f32),
            jax.ShapeDtypeStruct((n, 1), f32),
        ],
        scratch_shapes=[
            pltpu.VMEM((n, h), f32),
            pltpu.VMEM((1, h), f32),
        ],
    )(s3, W_gcn, a3, h12, csum_p, msk, wt, b3, a_s, bd, sb1, sb2a, sb2b)

    return jnp.concatenate([s_1, s_2a, s_2b], axis=0).reshape(1, 3 * n)
